# baseline (speedup 1.0000x reference)
import functools

import numpy as np
import jax
import jax.numpy as jnp
from jax import lax
from jax.experimental import pallas as pl
from jax.experimental.pallas import tpu as pltpu
from jax.experimental.pallas import tpu_sc as plsc

_N_HEADS = 16
_WS = 14
_L = _WS * _WS + 1
_N_REL = (2 * _WS - 1) ** 2 + 3
_LP = 208
_LR = 200
_ROWS_PER_TILE = 104
_R0_STEP = 96


def _rel_pos_ind_np() -> np.ndarray:
    ws = _WS
    coords = np.stack(np.meshgrid(np.arange(ws), np.arange(ws), indexing="ij"))
    coords = coords.reshape(2, -1)
    rel = coords[:, :, None] - coords[:, None, :]
    rel = np.transpose(rel, (1, 2, 0)).astype(np.int64)
    rel[:, :, 0] += ws - 1
    rel[:, :, 1] += ws - 1
    rel[:, :, 0] *= 2 * ws - 1
    area = ws * ws
    ind = np.zeros((area + 1, area + 1), dtype=np.int64)
    ind[1:, 1:] = rel.sum(-1)
    ind[0, :] = _N_REL - 3
    ind[:, 0] = _N_REL - 2
    ind[0, 0] = _N_REL - 1
    return ind.astype(np.int32)


_IND_PAD = np.zeros((_LR, _LP), dtype=np.int32)
_IND_PAD[:_L, :_L] = _rel_pos_ind_np()


@functools.lru_cache(maxsize=None)
def _make_bias_kernel():
    mesh = plsc.VectorSubcoreMesh(core_axis_name="c", subcore_axis_name="s")

    @functools.partial(
        pl.kernel,
        mesh=mesh,
        out_type=jax.ShapeDtypeStruct((_N_HEADS, _LR, _LP), jnp.float32),
        compiler_params=pltpu.CompilerParams(needs_layout_passes=False),
        scratch_types=[
            pltpu.VMEM((_N_REL * _N_HEADS,), jnp.float32),
            pltpu.VMEM((_ROWS_PER_TILE, _LP), jnp.int32),
            pltpu.VMEM((_ROWS_PER_TILE, _LP), jnp.float32),
        ],
    )
    def bias_kernel(table_hbm, ind_hbm, out_hbm, table_v, ind_v, out_v):
        h = lax.axis_index("s")
        half = lax.axis_index("c")
        r0 = half * _R0_STEP
        pltpu.sync_copy(table_hbm, table_v)
        pltpu.sync_copy(ind_hbm.at[pl.ds(r0, _ROWS_PER_TILE), :], ind_v)
        hvec = jnp.full((16,), h, dtype=jnp.int32)

        def row_body(r, carry):
            for c in range(_LP // 16):
                idx = ind_v[r, pl.ds(c * 16, 16)] * _N_HEADS + hvec
                out_v[r, pl.ds(c * 16, 16)] = plsc.load_gather(
                    table_v, [idx]
                )
            return carry

        lax.fori_loop(0, _ROWS_PER_TILE, row_body, 0)
        pltpu.sync_copy(out_v, out_hbm.at[h, pl.ds(r0, _ROWS_PER_TILE), :])

    return bias_kernel


def _add_body(bias_ref, in_ref, out_ref):
    out_ref[...] = in_ref[...] + bias_ref[:, : _L, : _L]


def _rel_add(bias, input):
    batch, n_heads = input.shape[0], input.shape[1]
    return pl.pallas_call(
        _add_body,
        grid=(n_heads, batch),
        in_specs=[
            pl.BlockSpec((1, _LR, _LP), lambda h, b: (h, 0, 0)),
            pl.BlockSpec((1, 1, _L, _L), lambda h, b: (b, h, 0, 0)),
        ],
        out_specs=pl.BlockSpec((1, 1, _L, _L), lambda h, b: (b, h, 0, 0)),
        out_shape=jax.ShapeDtypeStruct(input.shape, input.dtype),
        compiler_params=pltpu.CompilerParams(
            dimension_semantics=("arbitrary", "arbitrary"),
        ),
    )(bias, input)


def kernel(input, rel_pos_table):
    bias = _make_bias_kernel()(
        rel_pos_table.reshape(-1), jnp.asarray(_IND_PAD)
    )
    return _rel_add(bias, input)


# speedup vs baseline: 1.9198x; 1.9198x over previous
import functools

import numpy as np
import jax
import jax.numpy as jnp
from jax import lax
from jax.experimental import pallas as pl
from jax.experimental.pallas import tpu as pltpu
from jax.experimental.pallas import tpu_sc as plsc

_N_HEADS = 16
_WS = 14
_L = _WS * _WS + 1
_N_REL = (2 * _WS - 1) ** 2 + 3
_LP = 208
_LR = 200
_ROWS_PER_TILE = 104
_R0_STEP = 96


def _rel_pos_ind_np() -> np.ndarray:
    ws = _WS
    coords = np.stack(np.meshgrid(np.arange(ws), np.arange(ws), indexing="ij"))
    coords = coords.reshape(2, -1)
    rel = coords[:, :, None] - coords[:, None, :]
    rel = np.transpose(rel, (1, 2, 0)).astype(np.int64)
    rel[:, :, 0] += ws - 1
    rel[:, :, 1] += ws - 1
    rel[:, :, 0] *= 2 * ws - 1
    area = ws * ws
    ind = np.zeros((area + 1, area + 1), dtype=np.int64)
    ind[1:, 1:] = rel.sum(-1)
    ind[0, :] = _N_REL - 3
    ind[:, 0] = _N_REL - 2
    ind[0, 0] = _N_REL - 1
    return ind.astype(np.int32)


_IND_PAD = np.zeros((_LR, _LP), dtype=np.int32)
_IND_PAD[:_L, :_L] = _rel_pos_ind_np()


@functools.lru_cache(maxsize=None)
def _make_bias_kernel():
    mesh = plsc.VectorSubcoreMesh(core_axis_name="c", subcore_axis_name="s")

    @functools.partial(
        pl.kernel,
        mesh=mesh,
        out_type=jax.ShapeDtypeStruct((_N_HEADS, _LR, _LP), jnp.float32),
        compiler_params=pltpu.CompilerParams(needs_layout_passes=False),
        scratch_types=[
            pltpu.VMEM((_N_REL * _N_HEADS,), jnp.float32),
            pltpu.VMEM((_ROWS_PER_TILE, _LP), jnp.int32),
            pltpu.VMEM((_ROWS_PER_TILE, _LP), jnp.float32),
        ],
    )
    def bias_kernel(table_hbm, ind_hbm, out_hbm, table_v, ind_v, out_v):
        h = lax.axis_index("s")
        half = lax.axis_index("c")
        r0 = half * _R0_STEP
        pltpu.sync_copy(table_hbm, table_v)
        pltpu.sync_copy(ind_hbm.at[pl.ds(r0, _ROWS_PER_TILE), :], ind_v)
        hvec = jnp.full((16,), h, dtype=jnp.int32)

        def row_body(r, carry):
            for c in range(_LP // 16):
                idx = ind_v[r, pl.ds(c * 16, 16)] * _N_HEADS + hvec
                out_v[r, pl.ds(c * 16, 16)] = plsc.load_gather(
                    table_v, [idx]
                )
            return carry

        lax.fori_loop(0, _ROWS_PER_TILE, row_body, 0)
        pltpu.sync_copy(out_v, out_hbm.at[h, pl.ds(r0, _ROWS_PER_TILE), :])

    return bias_kernel


def _add_body(bias_ref, in_ref, out_ref):
    out_ref[...] = in_ref[...] + bias_ref[:, : _L, : _L]


def _rel_add(bias, input):
    batch, n_heads = input.shape[0], input.shape[1]
    return pl.pallas_call(
        _add_body,
        grid=(batch,),
        in_specs=[
            pl.BlockSpec((n_heads, _LR, _LP), lambda b: (0, 0, 0)),
            pl.BlockSpec((1, n_heads, _L, _L), lambda b: (b, 0, 0, 0)),
        ],
        out_specs=pl.BlockSpec((1, n_heads, _L, _L), lambda b: (b, 0, 0, 0)),
        out_shape=jax.ShapeDtypeStruct(input.shape, input.dtype),
        compiler_params=pltpu.CompilerParams(
            dimension_semantics=("arbitrary",),
        ),
    )(bias, input)


def kernel(input, rel_pos_table):
    bias = _make_bias_kernel()(
        rel_pos_table.reshape(-1), jnp.asarray(_IND_PAD)
    )
    return _rel_add(bias, input)


# speedup vs baseline: 5.6010x; 2.9176x over previous
import functools

import numpy as np
import jax
import jax.numpy as jnp
from jax import lax
from jax.experimental import pallas as pl
from jax.experimental.pallas import tpu as pltpu
from jax.experimental.pallas import tpu_sc as plsc

_N_HEADS = 16
_WS = 14
_L = _WS * _WS + 1
_N_REL = (2 * _WS - 1) ** 2 + 3
_LP = 208
_NI = 7
_N_TILES = 32


def _rel_pos_ind_np() -> np.ndarray:
    ws = _WS
    coords = np.stack(np.meshgrid(np.arange(ws), np.arange(ws), indexing="ij"))
    coords = coords.reshape(2, -1)
    rel = coords[:, :, None] - coords[:, None, :]
    rel = np.transpose(rel, (1, 2, 0)).astype(np.int64)
    rel[:, :, 0] += ws - 1
    rel[:, :, 1] += ws - 1
    rel[:, :, 0] *= 2 * ws - 1
    area = ws * ws
    ind = np.zeros((area + 1, area + 1), dtype=np.int64)
    ind[1:, 1:] = rel.sum(-1)
    ind[0, :] = _N_REL - 3
    ind[:, 0] = _N_REL - 2
    ind[0, 0] = _N_REL - 1
    return ind.astype(np.int32)


def _flat_gather_idx_np() -> np.ndarray:
    ind = np.zeros((_L, _LP), dtype=np.int64)
    ind[:, :_L] = _rel_pos_ind_np()
    h = np.arange(_N_HEADS, dtype=np.int64)
    fidx = ind[:, None, :] * _N_HEADS + h[None, :, None]
    return fidx.astype(np.int32)


_FIDX = _flat_gather_idx_np()


@functools.lru_cache(maxsize=None)
def _make_bias_kernel():
    mesh = plsc.VectorSubcoreMesh(core_axis_name="c", subcore_axis_name="s")

    @functools.partial(
        pl.kernel,
        mesh=mesh,
        out_type=jax.ShapeDtypeStruct((_L, _N_HEADS, _LP), jnp.float32),
        compiler_params=pltpu.CompilerParams(needs_layout_passes=False),
        scratch_types=[
            pltpu.VMEM((_N_REL * _N_HEADS,), jnp.float32),
            pltpu.VMEM((_NI, _N_HEADS, _LP), jnp.int32),
            pltpu.VMEM((_NI, _N_HEADS, _LP), jnp.float32),
        ],
    )
    def bias_kernel(table_hbm, fidx_hbm, out_hbm, table_v, idx_v, out_v):
        tid = lax.axis_index("s") * 2 + lax.axis_index("c")
        i0 = jnp.minimum(tid * _NI, _L - _NI)
        pltpu.sync_copy(table_hbm, table_v)
        pltpu.sync_copy(fidx_hbm.at[pl.ds(i0, _NI), :, :], idx_v)

        def i_body(i, carry):
            def h_body(h, carry2):
                for c in range(_LP // 16):
                    idx = idx_v[i, h, pl.ds(c * 16, 16)]
                    out_v[i, h, pl.ds(c * 16, 16)] = plsc.load_gather(
                        table_v, [idx]
                    )
                return carry2

            return lax.fori_loop(0, _N_HEADS, h_body, carry)

        lax.fori_loop(0, _NI, i_body, 0)
        pltpu.sync_copy(out_v, out_hbm.at[pl.ds(i0, _NI), :, :])

    return bias_kernel


def _add_body(bias_ref, in_ref, out_ref):
    out_ref[...] = in_ref[...] + bias_ref[:, :, : _L]


def _rel_add(bias, in_t):
    batch, n_heads = in_t.shape[0], in_t.shape[2]
    return pl.pallas_call(
        _add_body,
        grid=(batch,),
        in_specs=[
            pl.BlockSpec((_L, n_heads, _LP), lambda b: (0, 0, 0)),
            pl.BlockSpec((1, _L, n_heads, _L), lambda b: (b, 0, 0, 0)),
        ],
        out_specs=pl.BlockSpec((1, _L, n_heads, _L), lambda b: (b, 0, 0, 0)),
        out_shape=jax.ShapeDtypeStruct(in_t.shape, in_t.dtype),
        compiler_params=pltpu.CompilerParams(
            dimension_semantics=("arbitrary",),
        ),
    )(bias, in_t)


def kernel(input, rel_pos_table):
    bias = _make_bias_kernel()(
        rel_pos_table.reshape(-1), jnp.asarray(_FIDX)
    )
    in_t = jnp.transpose(input, (0, 2, 1, 3))
    out_t = _rel_add(bias, in_t)
    return jnp.transpose(out_t, (0, 2, 1, 3))


# speedup vs baseline: 6.3001x; 1.1248x over previous
import functools

import numpy as np
import jax
import jax.numpy as jnp
from jax import lax
from jax.experimental import pallas as pl
from jax.experimental.pallas import tpu as pltpu
from jax.experimental.pallas import tpu_sc as plsc

_N_HEADS = 16
_WS = 14
_L = _WS * _WS + 1
_N_REL = (2 * _WS - 1) ** 2 + 3
_LP = 208
_NI = 7
_N_TILES = 32


def _rel_pos_ind_np() -> np.ndarray:
    ws = _WS
    coords = np.stack(np.meshgrid(np.arange(ws), np.arange(ws), indexing="ij"))
    coords = coords.reshape(2, -1)
    rel = coords[:, :, None] - coords[:, None, :]
    rel = np.transpose(rel, (1, 2, 0)).astype(np.int64)
    rel[:, :, 0] += ws - 1
    rel[:, :, 1] += ws - 1
    rel[:, :, 0] *= 2 * ws - 1
    area = ws * ws
    ind = np.zeros((area + 1, area + 1), dtype=np.int64)
    ind[1:, 1:] = rel.sum(-1)
    ind[0, :] = _N_REL - 3
    ind[:, 0] = _N_REL - 2
    ind[0, 0] = _N_REL - 1
    return ind.astype(np.int32)


def _ind_pad_np() -> np.ndarray:
    ind = np.zeros((_L, 1, _LP), dtype=np.int32)
    ind[:, 0, :_L] = _rel_pos_ind_np()
    return ind


_IND_PAD = _ind_pad_np()


@functools.lru_cache(maxsize=None)
def _make_bias_kernel():
    mesh = plsc.VectorSubcoreMesh(core_axis_name="c", subcore_axis_name="s")

    @functools.partial(
        pl.kernel,
        mesh=mesh,
        out_type=jax.ShapeDtypeStruct((_L, _N_HEADS, _LP), jnp.float32),
        compiler_params=pltpu.CompilerParams(needs_layout_passes=False),
        scratch_types=[
            pltpu.VMEM((_N_REL * _N_HEADS,), jnp.float32),
            pltpu.VMEM((_NI, 1, _LP), jnp.int32),
            pltpu.VMEM((_NI, _N_HEADS, _LP), jnp.float32),
            pltpu.SemaphoreType.DMA,
            pltpu.SemaphoreType.DMA,
        ],
    )
    def bias_kernel(table_hbm, ind_hbm, out_hbm, table_v, ind_v, out_v,
                    sem_t, sem_i):
        tid = lax.axis_index("s") * 2 + lax.axis_index("c")
        i0 = jnp.minimum(tid * _NI, _L - _NI)
        cp_t = pltpu.async_copy(table_hbm, table_v, sem_t)
        cp_i = pltpu.async_copy(ind_hbm.at[pl.ds(i0, _NI), :, :], ind_v, sem_i)
        cp_t.wait()
        cp_i.wait()

        def i_body(i, carry):
            bases = [
                ind_v[i, 0, pl.ds(c * 16, 16)] * _N_HEADS
                for c in range(_LP // 16)
            ]

            @functools.partial(plsc.parallel_loop, 0, _N_HEADS, unroll=4)
            def h_body(h):
                for c in range(_LP // 16):
                    out_v[i, h, pl.ds(c * 16, 16)] = plsc.load_gather(
                        table_v, [bases[c] + h]
                    )

            return carry

        lax.fori_loop(0, _NI, i_body, 0)
        pltpu.sync_copy(out_v, out_hbm.at[pl.ds(i0, _NI), :, :])

    return bias_kernel


def _add_body(bias_ref, in_ref, out_ref):
    out_ref[...] = in_ref[...] + bias_ref[:, :, : _L]


def _rel_add(bias, in_t):
    batch, n_heads = in_t.shape[0], in_t.shape[2]
    bb = 2
    return pl.pallas_call(
        _add_body,
        grid=(batch // bb,),
        in_specs=[
            pl.BlockSpec((_L, n_heads, _LP), lambda b: (0, 0, 0)),
            pl.BlockSpec((bb, _L, n_heads, _L), lambda b: (b, 0, 0, 0)),
        ],
        out_specs=pl.BlockSpec((bb, _L, n_heads, _L), lambda b: (b, 0, 0, 0)),
        out_shape=jax.ShapeDtypeStruct(in_t.shape, in_t.dtype),
        compiler_params=pltpu.CompilerParams(
            dimension_semantics=("arbitrary",),
        ),
    )(bias, in_t)


def kernel(input, rel_pos_table):
    bias = _make_bias_kernel()(
        rel_pos_table.reshape(-1), jnp.asarray(_IND_PAD)
    )
    in_t = jnp.transpose(input, (0, 2, 1, 3))
    out_t = _rel_add(bias, in_t)
    return jnp.transpose(out_t, (0, 2, 1, 3))


# speedup vs baseline: 6.3487x; 1.0077x over previous
import functools

import numpy as np
import jax
import jax.numpy as jnp
from jax import lax
from jax.experimental import pallas as pl
from jax.experimental.pallas import tpu as pltpu
from jax.experimental.pallas import tpu_sc as plsc

_N_HEADS = 16
_WS = 14
_L = _WS * _WS + 1
_N_REL = (2 * _WS - 1) ** 2 + 3
_LP = 208
_NI = 7
_N_TILES = 32


def _rel_pos_ind_np() -> np.ndarray:
    ws = _WS
    coords = np.stack(np.meshgrid(np.arange(ws), np.arange(ws), indexing="ij"))
    coords = coords.reshape(2, -1)
    rel = coords[:, :, None] - coords[:, None, :]
    rel = np.transpose(rel, (1, 2, 0)).astype(np.int64)
    rel[:, :, 0] += ws - 1
    rel[:, :, 1] += ws - 1
    rel[:, :, 0] *= 2 * ws - 1
    area = ws * ws
    ind = np.zeros((area + 1, area + 1), dtype=np.int64)
    ind[1:, 1:] = rel.sum(-1)
    ind[0, :] = _N_REL - 3
    ind[:, 0] = _N_REL - 2
    ind[0, 0] = _N_REL - 1
    return ind.astype(np.int32)


def _ind_pad_np() -> np.ndarray:
    ind = np.zeros((_L, 1, _LP), dtype=np.int32)
    ind[:, 0, :_L] = _rel_pos_ind_np()
    return ind


_IND_PAD = _ind_pad_np()


@functools.lru_cache(maxsize=None)
def _make_bias_kernel():
    mesh = plsc.VectorSubcoreMesh(core_axis_name="c", subcore_axis_name="s")

    @functools.partial(
        pl.kernel,
        mesh=mesh,
        out_type=jax.ShapeDtypeStruct((_L, _N_HEADS, _LP), jnp.float32),
        compiler_params=pltpu.CompilerParams(needs_layout_passes=False),
        scratch_types=[
            pltpu.VMEM((_N_REL * _N_HEADS,), jnp.float32),
            pltpu.VMEM((_NI, 1, _LP), jnp.int32),
            pltpu.VMEM((_NI, _N_HEADS, _LP), jnp.float32),
            pltpu.SemaphoreType.DMA,
            pltpu.SemaphoreType.DMA,
        ],
    )
    def bias_kernel(table_hbm, ind_hbm, out_hbm, table_v, ind_v, out_v,
                    sem_t, sem_i):
        tid = lax.axis_index("s") * 2 + lax.axis_index("c")
        i0 = jnp.minimum(tid * _NI, _L - _NI)
        cp_t = pltpu.async_copy(table_hbm, table_v, sem_t)
        cp_i = pltpu.async_copy(ind_hbm.at[pl.ds(i0, _NI), :, :], ind_v, sem_i)
        cp_t.wait()
        cp_i.wait()

        def i_body(i, carry):
            bases = [
                ind_v[i, 0, pl.ds(c * 16, 16)] * _N_HEADS
                for c in range(_LP // 16)
            ]

            @functools.partial(plsc.parallel_loop, 0, _N_HEADS, unroll=4)
            def h_body(h):
                for c in range(_LP // 16):
                    out_v[i, h, pl.ds(c * 16, 16)] = plsc.load_gather(
                        table_v, [bases[c] + h]
                    )

            return carry

        lax.fori_loop(0, _NI, i_body, 0)
        pltpu.sync_copy(out_v, out_hbm.at[pl.ds(i0, _NI), :, :])

    return bias_kernel


def _add_body(bias_ref, in_ref, out_ref):
    out_ref[...] = in_ref[...] + bias_ref[:, :, : _L]


def _rel_add(bias, in_t):
    batch, n_heads = in_t.shape[0], in_t.shape[2]
    bb = 4
    return pl.pallas_call(
        _add_body,
        grid=(batch // bb,),
        in_specs=[
            pl.BlockSpec((_L, n_heads, _LP), lambda b: (0, 0, 0)),
            pl.BlockSpec((bb, _L, n_heads, _L), lambda b: (b, 0, 0, 0)),
        ],
        out_specs=pl.BlockSpec((bb, _L, n_heads, _L), lambda b: (b, 0, 0, 0)),
        out_shape=jax.ShapeDtypeStruct(in_t.shape, in_t.dtype),
        compiler_params=pltpu.CompilerParams(
            dimension_semantics=("arbitrary",),
            vmem_limit_bytes=112 * 1024 * 1024,
        ),
    )(bias, in_t)


def kernel(input, rel_pos_table):
    bias = _make_bias_kernel()(
        rel_pos_table.reshape(-1), jnp.asarray(_IND_PAD)
    )
    in_t = jnp.transpose(input, (0, 2, 1, 3))
    out_t = _rel_add(bias, in_t)
    return jnp.transpose(out_t, (0, 2, 1, 3))
